# Initial kernel scaffold; baseline (speedup 1.0000x reference)
import jax
import jax.numpy as jnp
from jax.experimental import pallas as pl


def kernel(x, edge_index, batch, params):
    raise NotImplementedError("write your pallas kernel here")


# speedup vs baseline: 1.0377x; 1.0377x over previous
import math

import jax
import jax.numpy as jnp
from jax.experimental import pallas as pl


def _copy_body(x_ref, o_ref):
    o_ref[...] = x_ref[...]


def _gconv(x, src, dst, valid, Wrel, brel, Wroot):
    N = x.shape[0]
    msgs = x[jnp.where(valid, src, 0)] * valid[:, None].astype(x.dtype)
    agg = jax.ops.segment_sum(msgs, jnp.where(valid, dst, N), num_segments=N + 1)[:N]
    return agg @ Wrel.T + brel + x @ Wroot.T


def _bn_eval(x, g, b):
    return g * (x / jnp.sqrt(1.0 + 1e-5)) + b


def _topk_pool(x, src, dst, valid, p, ratio=0.5):
    N = x.shape[0]
    score = jnp.tanh((x @ p) / jnp.linalg.norm(p))
    k = int(math.ceil(ratio * N))
    vals, perm = jax.lax.top_k(score, k)
    x_new = x[perm] * vals[:, None]
    new_idx = jnp.full((N,), -1, dtype=src.dtype).at[perm.astype(src.dtype)].set(jnp.arange(k, dtype=src.dtype))
    src_n = new_idx[src]
    dst_n = new_idx[dst]
    valid_n = valid & (src_n >= 0) & (dst_n >= 0)
    return x_new, jnp.where(valid_n, src_n, 0), jnp.where(valid_n, dst_n, 0), valid_n


def _readout(x):
    return jnp.concatenate([jnp.max(x, axis=0, keepdims=True), jnp.mean(x, axis=0, keepdims=True)], axis=1)


def kernel(x, edge_index, batch, params):
    del batch
    x = pl.pallas_call(
        _copy_body, out_shape=jax.ShapeDtypeStruct(x.shape, x.dtype))(x)
    src = edge_index[0].astype(jnp.int32)
    dst = edge_index[1].astype(jnp.int32)
    valid = jnp.ones(src.shape, dtype=bool)
    reads = []
    for layer in params["convs"]:
        x = jax.nn.relu(_gconv(x, src, dst, valid, layer["Wrel"], layer["brel"], layer["Wroot"]))
        x = _bn_eval(x, layer["g"], layer["b"])
        x, src, dst, valid = _topk_pool(x, src, dst, valid, layer["p"])
        reads.append(_readout(x))
    h = reads[0] + reads[1] + reads[2] + reads[3] + reads[4]
    W1, b1 = params["lin1"]
    W2, b2 = params["lin2"]
    W3, b3 = params["lin3"]
    h = jax.nn.relu(h @ W1.T + b1)
    h = jax.nn.relu(h @ W2.T + b2)
    return jax.nn.log_softmax(h @ W3.T + b3, axis=-1)


# speedup vs baseline: 10.9363x; 10.5387x over previous
import functools
import math

import jax
import jax.numpy as jnp
import numpy as np
from jax import lax
from jax.experimental import pallas as pl
from jax.experimental.pallas import tpu as pltpu
from jax.experimental.pallas import tpu_sc as plsc

N0 = 10000
NPAD = 10240
GR, GC = 80, 128
F = 128

NSUB = 16
EBLK = 128
NB = 157
HALF = NPAD // 2
NDUMMY = 128
ACC_ROWS = HALF + NDUMMY
ZROWS = ACC_ROWS // NSUB
WROWS = HALF // NSUB


def _segsum_body(x_hbm, src_hbm, dst_hbm, out_hbm, src_v, dst_v, rows_v,
                 zbuf, shared, sem):
    c = lax.axis_index("c")
    s = lax.axis_index("s")
    pltpu.sync_copy(src_hbm.at[s], src_v)
    pltpu.sync_copy(dst_hbm.at[s], dst_v)

    def zloop(i, carry):
        zbuf[i // 8, pl.ds((i % 8) * 16, 16)] = jnp.zeros((16,), jnp.float32)
        return carry

    lax.fori_loop(0, (EBLK * F) // 16, zloop, 0)
    zbase = s * ZROWS
    pltpu.sync_copy(zbuf, shared.at[pl.ds(zbase, EBLK)])
    pltpu.sync_copy(zbuf, shared.at[pl.ds(zbase + EBLK, EBLK)])
    pltpu.sync_copy(zbuf.at[pl.ds(0, ZROWS - 2 * EBLK)],
                    shared.at[pl.ds(zbase + 2 * EBLK, ZROWS - 2 * EBLK)])

    half_lo = c * HALF
    lane = lax.broadcasted_iota(jnp.int32, (16,), 0)

    def route(i, carry):
        j = i // 8
        v = i % 8
        d = dst_v[j, pl.ds(v * 16, 16)]
        local = d - half_lo
        dummy = HALF + ((v * 16 + lane) & (NDUMMY - 1))
        ok = (local >= 0) & (local < HALF)
        dst_v[j, pl.ds(v * 16, 16)] = jnp.where(ok, local, dummy)
        return carry

    lax.fori_loop(0, NB * 8, route, 0)
    plsc.subcore_barrier()

    def blk(j, carry):
        pltpu.async_copy(x_hbm.at[src_v.at[j]], rows_v, sem).wait()
        pltpu.sync_copy(rows_v, shared.at[dst_v.at[j]], add=True)
        return carry

    lax.fori_loop(0, NB, blk, 0)
    plsc.subcore_barrier()
    pltpu.sync_copy(shared.at[pl.ds(s * WROWS, WROWS)],
                    out_hbm.at[pl.ds(half_lo + s * WROWS, WROWS)])


def _sc_segment_sum(x, src_p, dst_p):
    return pl.kernel(
        _segsum_body,
        out_type=jax.ShapeDtypeStruct((NPAD, F), jnp.float32),
        mesh=plsc.VectorSubcoreMesh(core_axis_name="c", subcore_axis_name="s"),
        scratch_types=[
            pltpu.VMEM((NB, EBLK), jnp.int32),
            pltpu.VMEM((NB, EBLK), jnp.int32),
            pltpu.VMEM((EBLK, F), jnp.float32),
            pltpu.VMEM((EBLK, F), jnp.float32),
            pltpu.VMEM_SHARED((ACC_ROWS, F), jnp.float32),
            pltpu.SemaphoreType.DMA,
        ],
    )(x, src_p, dst_p)


def _dense_body(x_ref, agg_ref, wrel_ref, wroot_ref, brel_ref, g_ref, b_ref,
                p_ref, xbn_ref, score_ref):
    x = x_ref[...]
    agg = agg_ref[...]
    dn = (((1,), (1,)), ((), ()))
    h = lax.dot_general(agg, wrel_ref[...], dn, preferred_element_type=jnp.float32)
    h = h + brel_ref[...]
    h = h + lax.dot_general(x, wroot_ref[...], dn, preferred_element_type=jnp.float32)
    h = jnp.maximum(h, 0.0)
    xbn = g_ref[...] * (h / jnp.sqrt(jnp.float32(1.0 + 1e-5))) + b_ref[...]
    xbn_ref[...] = xbn
    p = p_ref[...]
    norm = jnp.sqrt(jnp.sum(p * p))
    sc = lax.dot_general(xbn, p, dn, preferred_element_type=jnp.float32)
    score_ref[...] = jnp.tanh(sc / norm)


def _sortable_u32(s):
    b = lax.bitcast_convert_type(s, jnp.uint32)
    return jnp.where(b >= jnp.uint32(0x80000000), ~b, b | jnp.uint32(0x80000000))


def _select_body(need0, m_ref, hist_ref, sel_ref):
    g = m_ref[...] != 0.0
    sel = jnp.zeros((GR, GC), dtype=jnp.bool_)
    need = jnp.int32(need0)
    keys = [_sortable_u32(hist_ref[j]) for j in range(hist_ref.shape[0])]
    ridx = lax.broadcasted_iota(jnp.uint32, (GR, GC), 0)
    cidx = lax.broadcasted_iota(jnp.uint32, (GR, GC), 1)
    keys.append(~(ridx * jnp.uint32(GC) + cidx))
    for key in keys:
        def bit_step(it, cand, key=key):
            t = cand | jnp.left_shift(jnp.uint32(1), jnp.uint32(31) - it.astype(jnp.uint32))
            cnt = jnp.sum((g & (key >= t)).astype(jnp.int32))
            return jnp.where(cnt >= need, t, cand)

        cand = lax.fori_loop(0, 32, bit_step, jnp.uint32(0))
        eq = g & (key == cand)
        gt = g & (key > cand)
        n_gt = jnp.sum(gt.astype(jnp.int32))
        n_ge = n_gt + jnp.sum(eq.astype(jnp.int32))
        all_t = n_ge == need
        sel = sel | gt | (eq & all_t)
        need = need - n_gt - jnp.where(all_t, n_ge - n_gt, jnp.int32(0))
        g = eq & jnp.logical_not(all_t)
    sel_ref[...] = sel.astype(jnp.float32)


def _apply_body(k_i, xbn_ref, sel_ref, score_ref, xout_ref, ro_ref):
    sel = sel_ref[...]
    xn = xbn_ref[...] * (sel * score_ref[...])
    xout_ref[...] = xn
    neg = jnp.where(sel > 0.0, xn, -jnp.inf)
    rmax = jnp.max(neg, axis=0, keepdims=True)
    rmean = jnp.sum(xn, axis=0, keepdims=True) / jnp.float32(k_i)
    ro_ref[...] = jnp.zeros((8, 2 * F), jnp.float32)
    ro_ref[0:1, 0:F] = rmax
    ro_ref[0:1, F:2 * F] = rmean


def _head_body(r0_ref, r1_ref, r2_ref, r3_ref, r4_ref, w1_ref, b1_ref,
               w2_ref, b2_ref, w3_ref, b3_ref, out_ref):
    r = r0_ref[...] + r1_ref[...] + r2_ref[...] + r3_ref[...] + r4_ref[...]
    dn = (((1,), (1,)), ((), ()))
    h = jnp.maximum(lax.dot_general(r, w1_ref[...], dn,
                                    preferred_element_type=jnp.float32) + b1_ref[...], 0.0)
    h = jnp.maximum(lax.dot_general(h, w2_ref[...], dn,
                                    preferred_element_type=jnp.float32) + b2_ref[...], 0.0)
    z = lax.dot_general(h, w3_ref[...], dn,
                        preferred_element_type=jnp.float32) + b3_ref[...]
    zmax = jnp.max(z, axis=1, keepdims=True)
    lse = jnp.log(jnp.sum(jnp.exp(z - zmax), axis=1, keepdims=True))
    out_ref[...] = (z - zmax - lse)[0:1, :]


def _f32(shape):
    return jax.ShapeDtypeStruct(shape, jnp.float32)


def kernel(x, edge_index, batch, params):
    del batch
    x = jnp.pad(x, ((0, NPAD - N0), (0, 0)))
    src = edge_index[0].astype(jnp.int32)
    dst = edge_index[1].astype(jnp.int32)
    perm = jnp.argsort(dst, stable=True)
    src_s = src[perm]
    dst_s = dst[perm]
    npadslots = NSUB * NB * EBLK - src.shape[0]
    fill_src = N0 + (jnp.arange(npadslots, dtype=jnp.int32) % (NPAD - N0))
    fill_dst = jnp.full((npadslots,), NPAD, dtype=jnp.int32)
    src_p = jnp.concatenate([src_s, fill_src]).reshape(NSUB, NB, EBLK)
    dst_p = jnp.concatenate([dst_s, fill_dst]).reshape(NSUB, NB, EBLK)
    m_grid = (jnp.arange(NPAD, dtype=jnp.int32) < N0).astype(jnp.float32).reshape(GR, GC)
    hist = []
    reads = []
    n_i = N0
    for layer in params["convs"]:
        k_i = int(math.ceil(0.5 * n_i))
        agg = _sc_segment_sum(x, src_p, dst_p)
        xbn, score_col = pl.pallas_call(
            _dense_body,
            out_shape=(_f32((NPAD, F)), _f32((NPAD, 1))),
        )(x, agg, layer["Wrel"], layer["Wroot"], layer["brel"].reshape(1, F),
          layer["g"].reshape(1, F), layer["b"].reshape(1, F),
          layer["p"].reshape(1, F))
        score_grid = score_col.reshape(GR, GC)
        sel_grid = pl.pallas_call(
            functools.partial(_select_body, k_i),
            out_shape=_f32((GR, GC)),
        )(m_grid, jnp.stack([score_grid] + hist))
        sel_col = sel_grid.reshape(NPAD, 1)
        x, ro = pl.pallas_call(
            functools.partial(_apply_body, k_i),
            out_shape=(_f32((NPAD, F)), _f32((8, 2 * F))),
        )(xbn, sel_col, score_col)
        reads.append(ro)
        m_grid = sel_grid
        hist.insert(0, score_grid)
        n_i = k_i
    W1, b1 = params["lin1"]
    W2, b2 = params["lin2"]
    W3, b3 = params["lin3"]
    out = pl.pallas_call(
        _head_body,
        out_shape=_f32((1, W3.shape[0])),
    )(reads[0], reads[1], reads[2], reads[3], reads[4],
      W1, b1.reshape(1, -1), W2, b2.reshape(1, -1), W3, b3.reshape(1, -1))
    return out


# speedup vs baseline: 15.2003x; 1.3899x over previous
import functools
import math

import jax
import jax.numpy as jnp
import numpy as np
from jax import lax
from jax.experimental import pallas as pl
from jax.experimental.pallas import tpu as pltpu
from jax.experimental.pallas import tpu_sc as plsc

N0 = 10000
NPAD = 10240
GR, GC = 80, 128
F = 128

NSUB = 16
EBLK = 128
NB = 158
HALF = NPAD // 2
NDUMMY = 64
ACC_ROWS = HALF + NDUMMY
ZROWS = ACC_ROWS // NSUB
WROWS = HALF // NSUB


def _segsum_body(x_hbm, src_hbm, dst_hbm, out_hbm, src_v, dst_v, rows_a,
                 rows_b, zbuf, shared, sem_a, sem_b):
    c = lax.axis_index("c")
    s = lax.axis_index("s")
    pltpu.sync_copy(src_hbm.at[s], src_v)
    pltpu.sync_copy(dst_hbm.at[s], dst_v)

    def zloop(i, carry):
        zbuf[i // 8, pl.ds((i % 8) * 16, 16)] = jnp.zeros((16,), jnp.float32)
        return carry

    lax.fori_loop(0, (96 * F) // 16, zloop, 0)
    zbase = s * ZROWS
    pltpu.sync_copy(zbuf, shared.at[pl.ds(zbase, 96)])
    pltpu.sync_copy(zbuf, shared.at[pl.ds(zbase + 96, 96)])
    pltpu.sync_copy(zbuf, shared.at[pl.ds(zbase + 192, 96)])
    pltpu.sync_copy(zbuf.at[pl.ds(0, ZROWS - 288)],
                    shared.at[pl.ds(zbase + 288, ZROWS - 288)])

    half_lo = c * HALF
    lane = lax.broadcasted_iota(jnp.int32, (16,), 0)

    def route(i, carry):
        j = i // 8
        v = i % 8
        d = dst_v[j, pl.ds(v * 16, 16)]
        local = d - half_lo
        dummy = HALF + ((v * 16 + lane) & (NDUMMY - 1))
        ok = (local >= 0) & (local < HALF)
        dst_v[j, pl.ds(v * 16, 16)] = jnp.where(ok, local, dummy)
        return carry

    lax.fori_loop(0, NB * 8, route, 0)
    plsc.subcore_barrier()

    pltpu.async_copy(x_hbm.at[src_v.at[0]], rows_a, sem_a)

    def blk(t, carry):
        j0 = t * 2
        j1 = j0 + 1
        pltpu.async_copy(x_hbm.at[src_v.at[j1]], rows_b, sem_b)
        pltpu.make_async_copy(x_hbm.at[src_v.at[j0]], rows_a, sem_a).wait()
        pltpu.sync_copy(rows_a, shared.at[dst_v.at[j0]], add=True)
        jn = jnp.minimum(j0 + 2, NB - 2)
        pltpu.async_copy(x_hbm.at[src_v.at[jn]], rows_a, sem_a)
        pltpu.make_async_copy(x_hbm.at[src_v.at[j1]], rows_b, sem_b).wait()
        pltpu.sync_copy(rows_b, shared.at[dst_v.at[j1]], add=True)
        return carry

    lax.fori_loop(0, NB // 2, blk, 0)
    pltpu.make_async_copy(x_hbm.at[src_v.at[NB - 2]], rows_a, sem_a).wait()
    plsc.subcore_barrier()
    pltpu.sync_copy(shared.at[pl.ds(s * WROWS, WROWS)],
                    out_hbm.at[pl.ds(half_lo + s * WROWS, WROWS)])


def _sc_segment_sum(x, src_p, dst_p):
    return pl.kernel(
        _segsum_body,
        out_type=jax.ShapeDtypeStruct((NPAD, F), jnp.float32),
        mesh=plsc.VectorSubcoreMesh(core_axis_name="c", subcore_axis_name="s"),
        scratch_types=[
            pltpu.VMEM((NB, EBLK), jnp.int32),
            pltpu.VMEM((NB, EBLK), jnp.int32),
            pltpu.VMEM((EBLK, F), jnp.float32),
            pltpu.VMEM((EBLK, F), jnp.float32),
            pltpu.VMEM((96, F), jnp.float32),
            pltpu.VMEM_SHARED((ACC_ROWS, F), jnp.float32),
            pltpu.SemaphoreType.DMA,
            pltpu.SemaphoreType.DMA,
        ],
    )(x, src_p, dst_p)


def _dense_body(x_ref, agg_ref, wrel_ref, wroot_ref, brel_ref, g_ref, b_ref,
                p_ref, xbn_ref, score_ref):
    x = x_ref[...]
    agg = agg_ref[...]
    dn = (((1,), (1,)), ((), ()))
    h = lax.dot_general(agg, wrel_ref[...], dn, preferred_element_type=jnp.float32)
    h = h + brel_ref[...]
    h = h + lax.dot_general(x, wroot_ref[...], dn, preferred_element_type=jnp.float32)
    h = jnp.maximum(h, 0.0)
    xbn = g_ref[...] * (h / jnp.sqrt(jnp.float32(1.0 + 1e-5))) + b_ref[...]
    xbn_ref[...] = xbn
    p = p_ref[...]
    norm = jnp.sqrt(jnp.sum(p * p))
    sc = lax.dot_general(xbn, p, dn, preferred_element_type=jnp.float32)
    score_ref[...] = jnp.tanh(sc / norm)


def _sortable_u32(s):
    b = lax.bitcast_convert_type(s, jnp.uint32)
    return jnp.where(b >= jnp.uint32(0x80000000), ~b, b | jnp.uint32(0x80000000))


def _select_body(need0, m_ref, hist_ref, sel_ref):
    g = m_ref[...] != 0.0
    sel = jnp.zeros((GR, GC), dtype=jnp.bool_)
    need = jnp.int32(need0)
    keys = [_sortable_u32(hist_ref[j]) for j in range(hist_ref.shape[0])]
    ridx = lax.broadcasted_iota(jnp.uint32, (GR, GC), 0)
    cidx = lax.broadcasted_iota(jnp.uint32, (GR, GC), 1)
    keys.append(~(ridx * jnp.uint32(GC) + cidx))
    for key in keys:
        def bit_step(it, cand, key=key):
            t = cand | jnp.left_shift(jnp.uint32(1), jnp.uint32(31) - it.astype(jnp.uint32))
            cnt = jnp.sum((g & (key >= t)).astype(jnp.int32))
            return jnp.where(cnt >= need, t, cand)

        cand = lax.fori_loop(0, 32, bit_step, jnp.uint32(0))
        eq = g & (key == cand)
        gt = g & (key > cand)
        n_gt = jnp.sum(gt.astype(jnp.int32))
        n_ge = n_gt + jnp.sum(eq.astype(jnp.int32))
        all_t = n_ge == need
        sel = sel | gt | (eq & all_t)
        need = need - n_gt - jnp.where(all_t, n_ge - n_gt, jnp.int32(0))
        g = eq & jnp.logical_not(all_t)
    sel_ref[...] = sel.astype(jnp.float32)


def _apply_body(k_i, xbn_ref, sel_ref, score_ref, xout_ref, ro_ref):
    sel = sel_ref[...]
    xn = xbn_ref[...] * (sel * score_ref[...])
    xout_ref[...] = xn
    neg = jnp.where(sel > 0.0, xn, -jnp.inf)
    rmax = jnp.max(neg, axis=0, keepdims=True)
    rmean = jnp.sum(xn, axis=0, keepdims=True) / jnp.float32(k_i)
    ro_ref[...] = jnp.zeros((8, 2 * F), jnp.float32)
    ro_ref[0:1, 0:F] = rmax
    ro_ref[0:1, F:2 * F] = rmean


def _head_body(r0_ref, r1_ref, r2_ref, r3_ref, r4_ref, w1_ref, b1_ref,
               w2_ref, b2_ref, w3_ref, b3_ref, out_ref):
    r = r0_ref[...] + r1_ref[...] + r2_ref[...] + r3_ref[...] + r4_ref[...]
    dn = (((1,), (1,)), ((), ()))
    h = jnp.maximum(lax.dot_general(r, w1_ref[...], dn,
                                    preferred_element_type=jnp.float32) + b1_ref[...], 0.0)
    h = jnp.maximum(lax.dot_general(h, w2_ref[...], dn,
                                    preferred_element_type=jnp.float32) + b2_ref[...], 0.0)
    z = lax.dot_general(h, w3_ref[...], dn,
                        preferred_element_type=jnp.float32) + b3_ref[...]
    zmax = jnp.max(z, axis=1, keepdims=True)
    lse = jnp.log(jnp.sum(jnp.exp(z - zmax), axis=1, keepdims=True))
    out_ref[...] = (z - zmax - lse)[0:1, :]


def _f32(shape):
    return jax.ShapeDtypeStruct(shape, jnp.float32)


def kernel(x, edge_index, batch, params):
    del batch
    x = jnp.pad(x, ((0, NPAD - N0), (0, 0)))
    src = edge_index[0].astype(jnp.int32)
    dst = edge_index[1].astype(jnp.int32)
    perm = jnp.argsort(dst, stable=True)
    src_s = src[perm]
    dst_s = dst[perm]
    npadslots = NSUB * NB * EBLK - src.shape[0]
    fill_src = N0 + (jnp.arange(npadslots, dtype=jnp.int32) % (NPAD - N0))
    fill_dst = jnp.full((npadslots,), NPAD, dtype=jnp.int32)
    src_p = jnp.concatenate([src_s, fill_src]).reshape(NSUB, NB, EBLK)
    dst_p = jnp.concatenate([dst_s, fill_dst]).reshape(NSUB, NB, EBLK)
    m_grid = (jnp.arange(NPAD, dtype=jnp.int32) < N0).astype(jnp.float32).reshape(GR, GC)
    hist = []
    reads = []
    n_i = N0
    for layer in params["convs"]:
        k_i = int(math.ceil(0.5 * n_i))
        agg = _sc_segment_sum(x, src_p, dst_p)
        xbn, score_col = pl.pallas_call(
            _dense_body,
            out_shape=(_f32((NPAD, F)), _f32((NPAD, 1))),
        )(x, agg, layer["Wrel"], layer["Wroot"], layer["brel"].reshape(1, F),
          layer["g"].reshape(1, F), layer["b"].reshape(1, F),
          layer["p"].reshape(1, F))
        score_grid = score_col.reshape(GR, GC)
        sel_grid = pl.pallas_call(
            functools.partial(_select_body, k_i),
            out_shape=_f32((GR, GC)),
        )(m_grid, jnp.stack([score_grid] + hist))
        sel_col = sel_grid.reshape(NPAD, 1)
        x, ro = pl.pallas_call(
            functools.partial(_apply_body, k_i),
            out_shape=(_f32((NPAD, F)), _f32((8, 2 * F))),
        )(xbn, sel_col, score_col)
        reads.append(ro)
        m_grid = sel_grid
        hist.insert(0, score_grid)
        n_i = k_i
    W1, b1 = params["lin1"]
    W2, b2 = params["lin2"]
    W3, b3 = params["lin3"]
    out = pl.pallas_call(
        _head_body,
        out_shape=_f32((1, W3.shape[0])),
    )(reads[0], reads[1], reads[2], reads[3], reads[4],
      W1, b1.reshape(1, -1), W2, b2.reshape(1, -1), W3, b3.reshape(1, -1))
    return out
